# Initial kernel scaffold; baseline (speedup 1.0000x reference)
import jax
import jax.numpy as jnp
from jax.experimental import pallas as pl


def kernel(drug_x, drug_edge_index, drug_edge_attr, drug_graph_ind, fp_batch, prot_x, prot_edge_index, prot_edge_attr, prot_graph_ind, params):
    raise NotImplementedError("write your pallas kernel here")


# speedup vs baseline: 1.8239x; 1.8239x over previous
import functools

import jax
import jax.numpy as jnp
from jax import lax
from jax.experimental import pallas as pl
from jax.experimental.pallas import tpu as pltpu
from jax.experimental.pallas import tpu_sc as plsc

HID = 128
B = 256
N = 10000
E = 320000
NC, NS, L = 2, 16, 16
NW = NC * NS
CHUNK = 128
NCHUNK = 79
EPAD = NW * NCHUNK * CHUNK
NPAD = 10240
ROWS_PER_SUB = NPAD // NS
NEG_INF = float("-inf")


def _dot(a, b):
    return jnp.dot(a, b, preferred_element_type=jnp.float32)


NODES_PER_W = 320
CHUNKS = E // CHUNK


def _prop_body(x_hbm, src_hbm, dst_hbm, ea_hbm, offs_hbm, zeros_hbm, out_hbm,
               src_v, dst_v, ea_v, gbuf, offs_v, acc, sem):
    cid = lax.axis_index("c")
    sid = lax.axis_index("s")
    w = cid * NS + sid
    pltpu.sync_copy(zeros_hbm.at[pl.ds(sid * ROWS_PER_SUB, ROWS_PER_SUB)],
                    acc.at[pl.ds(sid * ROWS_PER_SUB, ROWS_PER_SUB)])
    pltpu.sync_copy(offs_hbm.at[w], offs_v)
    plsc.subcore_barrier()

    i16 = lax.broadcasted_iota(jnp.int32, (L,), 0)
    ov = offs_v[0]
    o0 = jnp.max(jnp.where(i16 == 0, ov, -1))
    o1 = jnp.max(jnp.where(i16 == 1, ov, -1))
    k0 = o0 // CHUNK
    k1 = (o1 + CHUNK - 1) // CHUNK

    def chunk_body(k, carry):
        pltpu.sync_copy(src_hbm.at[k], src_v)
        pltpu.sync_copy(dst_hbm.at[k], dst_v)
        pltpu.sync_copy(ea_hbm.at[k], ea_v)
        pltpu.async_copy(x_hbm.at[src_v.at[0]], gbuf, sem).wait()
        for g in range(CHUNK // L):
            ids = k * CHUNK + g * L + i16
            keep = (ids >= o0) & (ids < o1)
            ea_v[0, pl.ds(g * L, L)] = jnp.where(
                keep, ea_v[0, pl.ds(g * L, L)], 0.0)

        def row_body(r, c2):
            s = plsc.load_gather(ea_v, [jnp.zeros((L,), jnp.int32),
                                        jnp.full((L,), r, jnp.int32)])
            for c in range(HID // L):
                gbuf[r, pl.ds(c * L, L)] = gbuf[r, pl.ds(c * L, L)] * s
            return c2

        lax.fori_loop(0, CHUNK, row_body, 0)
        pltpu.sync_copy(gbuf, acc.at[dst_v.at[0]], add=True)
        return carry

    lax.fori_loop(k0, k1, chunk_body, 0)
    plsc.subcore_barrier()
    pltpu.sync_copy(acc.at[pl.ds(w * NODES_PER_W, NODES_PER_W)],
                    out_hbm.at[pl.ds(w * NODES_PER_W, NODES_PER_W)])


@functools.cache
def _make_propagate():
    return pl.kernel(
        _prop_body,
        out_type=jax.ShapeDtypeStruct((NPAD, HID), jnp.float32),
        mesh=plsc.VectorSubcoreMesh(core_axis_name="c", subcore_axis_name="s",
                                    num_cores=NC, num_subcores=NS),
        compiler_params=pltpu.CompilerParams(needs_layout_passes=False),
        scratch_types=[
            pltpu.VMEM((1, CHUNK), jnp.int32),
            pltpu.VMEM((1, CHUNK), jnp.int32),
            pltpu.VMEM((1, CHUNK), jnp.float32),
            pltpu.VMEM((CHUNK, HID), jnp.float32),
            pltpu.VMEM((1, L), jnp.int32),
            pltpu.VMEM_SHARED((NPAD, HID), jnp.float32),
            pltpu.SemaphoreType.DMA,
        ],
    )


def _propagate(x, src, dst, ea, offs, zeros):
    return _make_propagate()(x, src, dst, ea, offs, zeros)


def _fc_body(x_ref, w_ref, b_ref, o_ref):
    o_ref[...] = _dot(x_ref[...], w_ref[...]) + b_ref[...]


def _fc(x, w, b, blk_rows=1000):
    n, k = x.shape
    m = w.shape[1]
    return pl.pallas_call(
        _fc_body,
        grid=(n // blk_rows,),
        in_specs=[pl.BlockSpec((blk_rows, k), lambda i: (i, 0)),
                  pl.BlockSpec((k, m), lambda i: (0, 0)),
                  pl.BlockSpec((1, m), lambda i: (0, 0))],
        out_specs=pl.BlockSpec((blk_rows, m), lambda i: (i, 0)),
        out_shape=jax.ShapeDtypeStruct((n, m), jnp.float32),
    )(x, w, b.reshape(1, m))


EBLK = 2048


def _edge_body(at_ref, w1t_ref, b1_ref, w2t_ref, b2_ref, o_ref):
    h = jnp.maximum(_dot(w1t_ref[...], at_ref[...]) + b1_ref[...],
                    0.0)
    e = jax.nn.sigmoid(_dot(w2t_ref[...], h) + b2_ref[...])
    col = (pl.program_id(0) * EBLK
           + lax.broadcasted_iota(jnp.int32, (1, EBLK), 1))
    o_ref[...] = jnp.where(col < E, e, 0.0)


def _edge_mlp(attr, p):
    f = attr.shape[1]
    at = jnp.pad(attr, ((0, EPAD - E), (0, 0))).T
    ea = pl.pallas_call(
        _edge_body,
        grid=(EPAD // EBLK,),
        in_specs=[pl.BlockSpec((f, EBLK), lambda i: (0, i)),
                  pl.BlockSpec((32, f), lambda i: (0, 0)),
                  pl.BlockSpec((32, 1), lambda i: (0, 0)),
                  pl.BlockSpec((1, 32), lambda i: (0, 0)),
                  pl.BlockSpec((1, 1), lambda i: (0, 0))],
        out_specs=pl.BlockSpec((1, EBLK), lambda i: (0, i)),
        out_shape=jax.ShapeDtypeStruct((1, EPAD), jnp.float32),
    )(at, p['W1'].T, p['b1'].reshape(32, 1), p['W2'].T, p['b2'].reshape(1, 1))
    return ea.reshape(EPAD)[:E]


def _gin_post_body(p_ref, g1_ref, be1_ref, w_ref, b_ref, g2_ref, be2_ref,
                   o_ref):
    agg = p_ref[:N]
    mu = jnp.mean(agg, axis=0, keepdims=True)
    var = jnp.mean((agg - mu) ** 2, axis=0, keepdims=True)
    h = jnp.maximum(
        g1_ref[...] * (agg - mu) / jnp.sqrt(var + 1e-5) + be1_ref[...], 0.0)
    h2 = _dot(h, w_ref[...]) + b_ref[...]
    mu2 = jnp.mean(h2, axis=0, keepdims=True)
    var2 = jnp.mean((h2 - mu2) ** 2, axis=0, keepdims=True)
    o_ref[...] = jnp.maximum(
        g2_ref[...] * (h2 - mu2) / jnp.sqrt(var2 + 1e-5) + be2_ref[...], 0.0)


def _gin_post(part, p):
    r1 = lambda a: a.reshape(1, HID)
    return pl.pallas_call(
        _gin_post_body,
        out_shape=jax.ShapeDtypeStruct((N, HID), jnp.float32),
        compiler_params=pltpu.CompilerParams(vmem_limit_bytes=100 * 1024 * 1024),
    )(part, r1(p['g1']), r1(p['be1']), p['W'], r1(p['b']),
      r1(p['g2']), r1(p['be2']))


def _pool_body(x_ref, gcol_ref, grow_ref, o_ref):
    y = x_ref[...]
    g = gcol_ref[...]
    k = 1
    while k < N:
        y_sh = jnp.concatenate(
            [jnp.full((k, HID), NEG_INF, jnp.float32), y[:-k]], axis=0)
        g_sh = jnp.concatenate(
            [jnp.full((k, 1), -1, jnp.int32), g[:-k]], axis=0)
        y = jnp.where(g == g_sh, jnp.maximum(y, y_sh), y)
        k *= 2
    grow = grow_ref[...]
    nxt = jnp.concatenate(
        [grow[:, 1:], jnp.full((1, 1), -1, jnp.int32)], axis=1)
    is_last = grow != nxt
    iota_b = lax.broadcasted_iota(jnp.int32, (B, 1), 0)
    eq = grow == iota_b
    sel = (eq & is_last).astype(jnp.float32)
    out = jnp.dot(sel, y, preferred_element_type=jnp.float32,
                  precision=lax.Precision.HIGHEST)
    cnt = jnp.sum(eq.astype(jnp.float32), axis=1, keepdims=True)
    o_ref[...] = jnp.where(cnt > 0, out, NEG_INF)


def _pool(x, gind):
    gind = gind.astype(jnp.int32)
    return pl.pallas_call(
        _pool_body,
        out_shape=jax.ShapeDtypeStruct((B, HID), jnp.float32),
        compiler_params=pltpu.CompilerParams(vmem_limit_bytes=100 * 1024 * 1024),
    )(x, gind.reshape(N, 1), gind.reshape(1, N))


def _bn(x, g, b):
    mu = jnp.mean(x, axis=0, keepdims=True)
    var = jnp.mean((x - mu) ** 2, axis=0, keepdims=True)
    return g * (x - mu) / jnp.sqrt(var + 1e-5) + b


def _gelu(x):
    return 0.5 * x * (1.0 + lax.erf(x / jnp.sqrt(jnp.float32(2.0))))


def _head_body(d_ref, pr_ref, fp_ref,
               fw1, fb1, fg1, fbe1, fw2, fb2, fg2, fbe2,
               wih, bih, whh, bhh,
               mw1, mb1, mw2, mb2, mw3, mb3, o_ref):
    dot = lambda a, b: _dot(a, b)
    f = jnp.maximum(_bn(dot(fp_ref[...], fw1[...]) + fb1[...],
                        fg1[...], fbe1[...]), 0.0)
    f = jnp.maximum(_bn(dot(f, fw2[...]) + fb2[...],
                        fg2[...], fbe2[...]), 0.0)
    gi = dot(d_ref[...], wih[...]) + bih[...]
    gh = dot(f, whh[...]) + bhh[...]
    r = jax.nn.sigmoid(gi[:, :HID] + gh[:, :HID])
    z = jax.nn.sigmoid(gi[:, HID:2 * HID] + gh[:, HID:2 * HID])
    nn_ = jnp.tanh(gi[:, 2 * HID:] + r * gh[:, 2 * HID:])
    h = (1.0 - z) * nn_ + z * f
    out = jnp.concatenate([h, pr_ref[...]], axis=1)
    h1 = _gelu(dot(out, mw1[...]) + mb1[...])
    h2 = _gelu(dot(h1, mw2[...]) + mb2[...])
    o_ref[...] = dot(h2, mw3[...]) + mb3[...]


def _head(d, pr, fp, params):
    fm = params['fp_mlp']
    g = params['gru']
    m = params['mlp']
    r1 = lambda a: a.reshape(1, -1)
    return pl.pallas_call(
        _head_body,
        out_shape=jax.ShapeDtypeStruct((B, 1), jnp.float32),
        compiler_params=pltpu.CompilerParams(vmem_limit_bytes=100 * 1024 * 1024),
    )(d, pr, fp,
      fm['W1'], r1(fm['b1']), r1(fm['g1']), r1(fm['be1']),
      fm['W2'], r1(fm['b2']), r1(fm['g2']), r1(fm['be2']),
      g['W_ih'], r1(g['b_ih']), g['W_hh'], r1(g['b_hh']),
      m['W1'], r1(m['b1']), m['W2'], r1(m['b2']), m['W3'], r1(m['b3']))


def _gnn(x, edge_index, edge_attr, gind, fcW, fcb, edge_p, layers):
    xin = _fc(x, fcW, fcb)
    ea = _edge_mlp(edge_attr, edge_p)
    src = edge_index[0].astype(jnp.int32)
    dst = edge_index[1].astype(jnp.int32)
    perm = jnp.argsort(dst // NODES_PER_W, stable=True)
    src_s = src[perm].reshape(CHUNKS, 1, CHUNK)
    dst_s = dst[perm]
    ea_s = ea[perm].reshape(CHUNKS, 1, CHUNK)
    bounds = jnp.arange(0, NW * NODES_PER_W, NODES_PER_W, dtype=jnp.int32)
    offs = jnp.searchsorted(dst_s // NODES_PER_W,
                            jnp.arange(NW, dtype=jnp.int32)).astype(jnp.int32)
    offs_end = jnp.concatenate([offs[1:], jnp.array([E], jnp.int32)])
    offs2 = jnp.pad(jnp.stack([offs, offs_end], axis=1),
                    ((0, 0), (0, L - 2))).reshape(NW, 1, L)
    dst_s = dst_s.reshape(CHUNKS, 1, CHUNK)
    zeros = jnp.zeros((NPAD, HID), jnp.float32)
    for p in layers:
        agg = _propagate(xin, src_s, dst_s, ea_s, offs2, zeros)
        xin = _gin_post(agg, p)
    return _pool(xin, gind)


def kernel(drug_x, drug_edge_index, drug_edge_attr, drug_graph_ind, fp_batch,
           prot_x, prot_edge_index, prot_edge_attr, prot_graph_ind, params):
    d = _gnn(drug_x, drug_edge_index, drug_edge_attr, drug_graph_ind,
             params['mol_fc_W'], params['mol_fc_b'], params['mol_edge'],
             params['mol_gnn'])
    pr = _gnn(prot_x, prot_edge_index, prot_edge_attr, prot_graph_ind,
              params['prot_fc_W'], params['prot_fc_b'], params['prot_edge'],
              params['prot_gnn'])
    out = _head(d, pr, fp_batch, params)
    return out.reshape(B)


# speedup vs baseline: 2.2457x; 1.2313x over previous
import functools

import jax
import jax.numpy as jnp
from jax import lax
from jax.experimental import pallas as pl
from jax.experimental.pallas import tpu as pltpu
from jax.experimental.pallas import tpu_sc as plsc

HID = 128
B = 256
N = 10000
E = 320000
NC, NS, L = 2, 16, 16
NW = NC * NS
CHUNK = 128
NCHUNK = 79
EPAD = NW * NCHUNK * CHUNK
NPAD = 10240
ROWS_PER_SUB = NPAD // NS
NEG_INF = float("-inf")


def _dot(a, b):
    return jnp.dot(a, b, preferred_element_type=jnp.float32)


NODES_PER_W = 320
CHUNKS = E // CHUNK


def _prop_body(x_hbm, src_hbm, dst_hbm, ea_hbm, offs_hbm, zeros_hbm, out_hbm,
               src_v, dst_v, ea_v, gbuf, offs_v, acc, sem):
    cid = lax.axis_index("c")
    sid = lax.axis_index("s")
    w = cid * NS + sid
    pltpu.sync_copy(zeros_hbm.at[pl.ds(sid * ROWS_PER_SUB, ROWS_PER_SUB)],
                    acc.at[pl.ds(sid * ROWS_PER_SUB, ROWS_PER_SUB)])
    pltpu.sync_copy(offs_hbm.at[w], offs_v)
    plsc.subcore_barrier()

    i16 = lax.broadcasted_iota(jnp.int32, (L,), 0)
    ov = offs_v[0]
    o0 = jnp.max(jnp.where(i16 == 0, ov, -1))
    o1 = jnp.max(jnp.where(i16 == 1, ov, -1))
    k0 = o0 // CHUNK
    k1 = (o1 + CHUNK - 1) // CHUNK

    def load_idx(b, k):
        kc = jnp.minimum(k, CHUNKS - 1)
        pltpu.sync_copy(src_hbm.at[kc], src_v.at[b])
        pltpu.sync_copy(dst_hbm.at[kc], dst_v.at[b])
        pltpu.sync_copy(ea_hbm.at[kc], ea_v.at[b])

    def start_gather(b):
        pltpu.async_copy(x_hbm.at[src_v.at[b, 0]], gbuf.at[b], sem.at[b])

    def wait_gather(b):
        pltpu.make_async_copy(x_hbm.at[src_v.at[b, 0]], gbuf.at[b],
                              sem.at[b]).wait()

    def process(b, k):
        for g in range(CHUNK // L):
            ids = k * CHUNK + g * L + i16
            keep = (ids >= o0) & (ids < o1)
            ea_v[b, 0, pl.ds(g * L, L)] = jnp.where(
                keep, ea_v[b, 0, pl.ds(g * L, L)], 0.0)

        def row_body(r4, c2):
            for u in range(4):
                r = r4 * 4 + u
                s = plsc.load_gather(ea_v.at[b],
                                     [jnp.zeros((L,), jnp.int32),
                                      jnp.full((L,), r, jnp.int32)])
                for c in range(HID // L):
                    gbuf[b, r, pl.ds(c * L, L)] = (
                        gbuf[b, r, pl.ds(c * L, L)] * s)
            return c2

        lax.fori_loop(0, CHUNK // 4, row_body, 0)
        pltpu.sync_copy(gbuf.at[b], acc.at[dst_v.at[b, 0]], add=True)

    load_idx(0, k0)
    start_gather(0)

    def pair_body(t, carry):
        for bb in range(2):
            k = k0 + 2 * t + bb
            load_idx(1 - bb, k + 1)
            start_gather(1 - bb)
            wait_gather(bb)
            process(bb, k)
        return carry

    npairs = (k1 - k0 + 1) // 2
    lax.fori_loop(0, npairs, pair_body, 0)
    wait_gather(0)

    plsc.subcore_barrier()
    pltpu.sync_copy(acc.at[pl.ds(w * NODES_PER_W, NODES_PER_W)],
                    out_hbm.at[pl.ds(w * NODES_PER_W, NODES_PER_W)])


@functools.cache
def _make_propagate():
    return pl.kernel(
        _prop_body,
        out_type=jax.ShapeDtypeStruct((NPAD, HID), jnp.float32),
        mesh=plsc.VectorSubcoreMesh(core_axis_name="c", subcore_axis_name="s",
                                    num_cores=NC, num_subcores=NS),
        compiler_params=pltpu.CompilerParams(needs_layout_passes=False),
        scratch_types=[
            pltpu.VMEM((2, 1, CHUNK), jnp.int32),
            pltpu.VMEM((2, 1, CHUNK), jnp.int32),
            pltpu.VMEM((2, 1, CHUNK), jnp.float32),
            pltpu.VMEM((2, CHUNK, HID), jnp.float32),
            pltpu.VMEM((1, L), jnp.int32),
            pltpu.VMEM_SHARED((NPAD, HID), jnp.float32),
            pltpu.SemaphoreType.DMA((2,)),
        ],
    )


def _propagate(x, src, dst, ea, offs, zeros):
    return _make_propagate()(x, src, dst, ea, offs, zeros)


def _fc_body(x_ref, w_ref, b_ref, o_ref):
    o_ref[...] = _dot(x_ref[...], w_ref[...]) + b_ref[...]


def _fc(x, w, b, blk_rows=1000):
    n, k = x.shape
    m = w.shape[1]
    return pl.pallas_call(
        _fc_body,
        grid=(n // blk_rows,),
        in_specs=[pl.BlockSpec((blk_rows, k), lambda i: (i, 0)),
                  pl.BlockSpec((k, m), lambda i: (0, 0)),
                  pl.BlockSpec((1, m), lambda i: (0, 0))],
        out_specs=pl.BlockSpec((blk_rows, m), lambda i: (i, 0)),
        out_shape=jax.ShapeDtypeStruct((n, m), jnp.float32),
    )(x, w, b.reshape(1, m))


EBLK = 2048


def _edge_body(at_ref, w1t_ref, b1_ref, w2t_ref, b2_ref, o_ref):
    h = jnp.maximum(_dot(w1t_ref[...], at_ref[...]) + b1_ref[...],
                    0.0)
    e = jax.nn.sigmoid(_dot(w2t_ref[...], h) + b2_ref[...])
    col = (pl.program_id(0) * EBLK
           + lax.broadcasted_iota(jnp.int32, (1, EBLK), 1))
    o_ref[...] = jnp.where(col < E, e, 0.0)


def _edge_mlp(attr, p):
    f = attr.shape[1]
    at = jnp.pad(attr, ((0, EPAD - E), (0, 0))).T
    ea = pl.pallas_call(
        _edge_body,
        grid=(EPAD // EBLK,),
        in_specs=[pl.BlockSpec((f, EBLK), lambda i: (0, i)),
                  pl.BlockSpec((32, f), lambda i: (0, 0)),
                  pl.BlockSpec((32, 1), lambda i: (0, 0)),
                  pl.BlockSpec((1, 32), lambda i: (0, 0)),
                  pl.BlockSpec((1, 1), lambda i: (0, 0))],
        out_specs=pl.BlockSpec((1, EBLK), lambda i: (0, i)),
        out_shape=jax.ShapeDtypeStruct((1, EPAD), jnp.float32),
    )(at, p['W1'].T, p['b1'].reshape(32, 1), p['W2'].T, p['b2'].reshape(1, 1))
    return ea.reshape(EPAD)[:E]


def _gin_post_body(p_ref, g1_ref, be1_ref, w_ref, b_ref, g2_ref, be2_ref,
                   o_ref):
    agg = p_ref[:N]
    mu = jnp.mean(agg, axis=0, keepdims=True)
    var = jnp.mean((agg - mu) ** 2, axis=0, keepdims=True)
    h = jnp.maximum(
        g1_ref[...] * (agg - mu) / jnp.sqrt(var + 1e-5) + be1_ref[...], 0.0)
    h2 = _dot(h, w_ref[...]) + b_ref[...]
    mu2 = jnp.mean(h2, axis=0, keepdims=True)
    var2 = jnp.mean((h2 - mu2) ** 2, axis=0, keepdims=True)
    o_ref[...] = jnp.maximum(
        g2_ref[...] * (h2 - mu2) / jnp.sqrt(var2 + 1e-5) + be2_ref[...], 0.0)


def _gin_post(part, p):
    r1 = lambda a: a.reshape(1, HID)
    return pl.pallas_call(
        _gin_post_body,
        out_shape=jax.ShapeDtypeStruct((N, HID), jnp.float32),
        compiler_params=pltpu.CompilerParams(vmem_limit_bytes=100 * 1024 * 1024),
    )(part, r1(p['g1']), r1(p['be1']), p['W'], r1(p['b']),
      r1(p['g2']), r1(p['be2']))


def _pool_body(x_ref, gcol_ref, grow_ref, o_ref):
    y = x_ref[...]
    g = gcol_ref[...]
    k = 1
    while k < N:
        y_sh = jnp.concatenate(
            [jnp.full((k, HID), NEG_INF, jnp.float32), y[:-k]], axis=0)
        g_sh = jnp.concatenate(
            [jnp.full((k, 1), -1, jnp.int32), g[:-k]], axis=0)
        y = jnp.where(g == g_sh, jnp.maximum(y, y_sh), y)
        k *= 2
    grow = grow_ref[...]
    nxt = jnp.concatenate(
        [grow[:, 1:], jnp.full((1, 1), -1, jnp.int32)], axis=1)
    is_last = grow != nxt
    iota_b = lax.broadcasted_iota(jnp.int32, (B, 1), 0)
    eq = grow == iota_b
    sel = (eq & is_last).astype(jnp.float32)
    out = jnp.dot(sel, y, preferred_element_type=jnp.float32,
                  precision=lax.Precision.HIGHEST)
    cnt = jnp.sum(eq.astype(jnp.float32), axis=1, keepdims=True)
    o_ref[...] = jnp.where(cnt > 0, out, NEG_INF)


def _pool(x, gind):
    gind = gind.astype(jnp.int32)
    return pl.pallas_call(
        _pool_body,
        out_shape=jax.ShapeDtypeStruct((B, HID), jnp.float32),
        compiler_params=pltpu.CompilerParams(vmem_limit_bytes=100 * 1024 * 1024),
    )(x, gind.reshape(N, 1), gind.reshape(1, N))


def _bn(x, g, b):
    mu = jnp.mean(x, axis=0, keepdims=True)
    var = jnp.mean((x - mu) ** 2, axis=0, keepdims=True)
    return g * (x - mu) / jnp.sqrt(var + 1e-5) + b


def _gelu(x):
    return 0.5 * x * (1.0 + lax.erf(x / jnp.sqrt(jnp.float32(2.0))))


def _head_body(d_ref, pr_ref, fp_ref,
               fw1, fb1, fg1, fbe1, fw2, fb2, fg2, fbe2,
               wih, bih, whh, bhh,
               mw1, mb1, mw2, mb2, mw3, mb3, o_ref):
    dot = lambda a, b: _dot(a, b)
    f = jnp.maximum(_bn(dot(fp_ref[...], fw1[...]) + fb1[...],
                        fg1[...], fbe1[...]), 0.0)
    f = jnp.maximum(_bn(dot(f, fw2[...]) + fb2[...],
                        fg2[...], fbe2[...]), 0.0)
    gi = dot(d_ref[...], wih[...]) + bih[...]
    gh = dot(f, whh[...]) + bhh[...]
    r = jax.nn.sigmoid(gi[:, :HID] + gh[:, :HID])
    z = jax.nn.sigmoid(gi[:, HID:2 * HID] + gh[:, HID:2 * HID])
    nn_ = jnp.tanh(gi[:, 2 * HID:] + r * gh[:, 2 * HID:])
    h = (1.0 - z) * nn_ + z * f
    out = jnp.concatenate([h, pr_ref[...]], axis=1)
    h1 = _gelu(dot(out, mw1[...]) + mb1[...])
    h2 = _gelu(dot(h1, mw2[...]) + mb2[...])
    o_ref[...] = dot(h2, mw3[...]) + mb3[...]


def _head(d, pr, fp, params):
    fm = params['fp_mlp']
    g = params['gru']
    m = params['mlp']
    r1 = lambda a: a.reshape(1, -1)
    return pl.pallas_call(
        _head_body,
        out_shape=jax.ShapeDtypeStruct((B, 1), jnp.float32),
        compiler_params=pltpu.CompilerParams(vmem_limit_bytes=100 * 1024 * 1024),
    )(d, pr, fp,
      fm['W1'], r1(fm['b1']), r1(fm['g1']), r1(fm['be1']),
      fm['W2'], r1(fm['b2']), r1(fm['g2']), r1(fm['be2']),
      g['W_ih'], r1(g['b_ih']), g['W_hh'], r1(g['b_hh']),
      m['W1'], r1(m['b1']), m['W2'], r1(m['b2']), m['W3'], r1(m['b3']))


def _gnn(x, edge_index, edge_attr, gind, fcW, fcb, edge_p, layers):
    xin = _fc(x, fcW, fcb)
    ea = _edge_mlp(edge_attr, edge_p)
    src = edge_index[0].astype(jnp.int32)
    dst = edge_index[1].astype(jnp.int32)
    perm = jnp.argsort(dst // NODES_PER_W, stable=True)
    src_s = src[perm].reshape(CHUNKS, 1, CHUNK)
    dst_s = dst[perm]
    ea_s = ea[perm].reshape(CHUNKS, 1, CHUNK)
    bounds = jnp.arange(0, NW * NODES_PER_W, NODES_PER_W, dtype=jnp.int32)
    offs = jnp.searchsorted(dst_s // NODES_PER_W,
                            jnp.arange(NW, dtype=jnp.int32)).astype(jnp.int32)
    offs_end = jnp.concatenate([offs[1:], jnp.array([E], jnp.int32)])
    offs2 = jnp.pad(jnp.stack([offs, offs_end], axis=1),
                    ((0, 0), (0, L - 2))).reshape(NW, 1, L)
    dst_s = dst_s.reshape(CHUNKS, 1, CHUNK)
    zeros = jnp.zeros((NPAD, HID), jnp.float32)
    for p in layers:
        agg = _propagate(xin, src_s, dst_s, ea_s, offs2, zeros)
        xin = _gin_post(agg, p)
    return _pool(xin, gind)


def kernel(drug_x, drug_edge_index, drug_edge_attr, drug_graph_ind, fp_batch,
           prot_x, prot_edge_index, prot_edge_attr, prot_graph_ind, params):
    d = _gnn(drug_x, drug_edge_index, drug_edge_attr, drug_graph_ind,
             params['mol_fc_W'], params['mol_fc_b'], params['mol_edge'],
             params['mol_gnn'])
    pr = _gnn(prot_x, prot_edge_index, prot_edge_attr, prot_graph_ind,
              params['prot_fc_W'], params['prot_fc_b'], params['prot_edge'],
              params['prot_gnn'])
    out = _head(d, pr, fp_batch, params)
    return out.reshape(B)


# speedup vs baseline: 2.4029x; 1.0700x over previous
import functools

import jax
import jax.numpy as jnp
from jax import lax
from jax.experimental import pallas as pl
from jax.experimental.pallas import tpu as pltpu
from jax.experimental.pallas import tpu_sc as plsc

HID = 128
B = 256
N = 10000
E = 320000
NC, NS, L = 2, 16, 16
NW = NC * NS
CHUNK = 112
EPAD = 323584
NPAD = 10240
ROWS_PER_SUB = NPAD // NS
NEG_INF = float("-inf")


def _dot(a, b):
    return jnp.dot(a, b, preferred_element_type=jnp.float32)


NODES_PER_W = 320
CHUNKS = -(-E // CHUNK)
EPAD_SC = CHUNKS * CHUNK


def _prop_body(x_hbm, src_hbm, dst_hbm, ea_hbm, offs_hbm, zeros_hbm, out_hbm,
               src_v, dst_v, ea_v, gbuf, offs_v, acc, sem, sem2):
    cid = lax.axis_index("c")
    sid = lax.axis_index("s")
    w = cid * NS + sid
    pltpu.sync_copy(zeros_hbm.at[pl.ds(sid * ROWS_PER_SUB, ROWS_PER_SUB)],
                    acc.at[pl.ds(sid * ROWS_PER_SUB, ROWS_PER_SUB)])
    pltpu.sync_copy(offs_hbm.at[w], offs_v)
    plsc.subcore_barrier()

    i16 = lax.broadcasted_iota(jnp.int32, (L,), 0)
    ov = offs_v[0]
    o0 = jnp.max(jnp.where(i16 == 0, ov, -1))
    o1 = jnp.max(jnp.where(i16 == 1, ov, -1))
    k0 = o0 // CHUNK
    k1 = (o1 + CHUNK - 1) // CHUNK

    def load_idx(b, k):
        kc = jnp.minimum(k, CHUNKS - 1)
        pltpu.sync_copy(src_hbm.at[kc], src_v.at[b])
        pltpu.sync_copy(dst_hbm.at[kc], dst_v.at[b])
        pltpu.sync_copy(ea_hbm.at[kc], ea_v.at[b])

    def start_gather(b):
        pltpu.async_copy(x_hbm.at[src_v.at[b, 0]], gbuf.at[b], sem.at[b])

    def wait_gather(b):
        pltpu.make_async_copy(x_hbm.at[src_v.at[b, 0]], gbuf.at[b],
                              sem.at[b]).wait()

    def process(b, k):
        for g in range(CHUNK // L):
            ids = k * CHUNK + g * L + i16
            keep = (ids >= o0) & (ids < o1)
            ea_v[b, 0, pl.ds(g * L, L)] = jnp.where(
                keep, ea_v[b, 0, pl.ds(g * L, L)], 0.0)

        def row_body(r4, c2):
            for u in range(4):
                r = r4 * 4 + u
                s = plsc.load_gather(ea_v.at[b],
                                     [jnp.zeros((L,), jnp.int32),
                                      jnp.full((L,), r, jnp.int32)])
                for c in range(HID // L):
                    gbuf[b, r, pl.ds(c * L, L)] = (
                        gbuf[b, r, pl.ds(c * L, L)] * s)
            return c2

        lax.fori_loop(0, CHUNK // 4, row_body, 0)

    def start_scatter(b):
        pltpu.async_copy(gbuf.at[b], acc.at[dst_v.at[b, 0]], sem2, add=True)

    def wait_scatter(b):
        pltpu.make_async_copy(gbuf.at[b], acc.at[dst_v.at[b, 0]], sem2).wait()

    load_idx(0, k0)
    start_gather(0)

    def round_body(t, carry):
        for bb in range(3):
            k = k0 + 3 * t + bb
            bn = (bb + 1) % 3
            bp = (bb + 2) % 3
            load_idx(bn, k + 1)
            start_gather(bn)
            wait_gather(bb)
            process(bb, k)
            pl.when(k > k0)(lambda: wait_scatter(bp))
            start_scatter(bb)
        return carry

    nrounds = (k1 - k0 + 2) // 3
    lax.fori_loop(0, nrounds, round_body, 0)
    wait_gather(0)
    pl.when(k1 > k0)(lambda: wait_scatter(2))

    plsc.subcore_barrier()
    pltpu.sync_copy(acc.at[pl.ds(w * NODES_PER_W, NODES_PER_W)],
                    out_hbm.at[pl.ds(w * NODES_PER_W, NODES_PER_W)])


@functools.cache
def _make_propagate():
    return pl.kernel(
        _prop_body,
        out_type=jax.ShapeDtypeStruct((NPAD, HID), jnp.float32),
        mesh=plsc.VectorSubcoreMesh(core_axis_name="c", subcore_axis_name="s",
                                    num_cores=NC, num_subcores=NS),
        compiler_params=pltpu.CompilerParams(needs_layout_passes=False),
        scratch_types=[
            pltpu.VMEM((3, 1, CHUNK), jnp.int32),
            pltpu.VMEM((3, 1, CHUNK), jnp.int32),
            pltpu.VMEM((3, 1, CHUNK), jnp.float32),
            pltpu.VMEM((3, CHUNK, HID), jnp.float32),
            pltpu.VMEM((1, L), jnp.int32),
            pltpu.VMEM_SHARED((NPAD, HID), jnp.float32),
            pltpu.SemaphoreType.DMA((3,)),
            pltpu.SemaphoreType.DMA,
        ],
    )


def _propagate(x, src, dst, ea, offs, zeros):
    return _make_propagate()(x, src, dst, ea, offs, zeros)


def _fc_body(x_ref, w_ref, b_ref, o_ref):
    o_ref[...] = _dot(x_ref[...], w_ref[...]) + b_ref[...]


def _fc(x, w, b, blk_rows=1000):
    n, k = x.shape
    m = w.shape[1]
    return pl.pallas_call(
        _fc_body,
        grid=(n // blk_rows,),
        in_specs=[pl.BlockSpec((blk_rows, k), lambda i: (i, 0)),
                  pl.BlockSpec((k, m), lambda i: (0, 0)),
                  pl.BlockSpec((1, m), lambda i: (0, 0))],
        out_specs=pl.BlockSpec((blk_rows, m), lambda i: (i, 0)),
        out_shape=jax.ShapeDtypeStruct((n, m), jnp.float32),
    )(x, w, b.reshape(1, m))


EBLK = 2048


def _edge_body(at_ref, w1t_ref, b1_ref, w2t_ref, b2_ref, o_ref):
    h = jnp.maximum(_dot(w1t_ref[...], at_ref[...]) + b1_ref[...],
                    0.0)
    e = jax.nn.sigmoid(_dot(w2t_ref[...], h) + b2_ref[...])
    col = (pl.program_id(0) * EBLK
           + lax.broadcasted_iota(jnp.int32, (1, EBLK), 1))
    o_ref[...] = jnp.where(col < E, e, 0.0)


def _edge_mlp(attr, p):
    f = attr.shape[1]
    at = jnp.pad(attr, ((0, EPAD - E), (0, 0))).T
    ea = pl.pallas_call(
        _edge_body,
        grid=(EPAD // EBLK,),
        in_specs=[pl.BlockSpec((f, EBLK), lambda i: (0, i)),
                  pl.BlockSpec((32, f), lambda i: (0, 0)),
                  pl.BlockSpec((32, 1), lambda i: (0, 0)),
                  pl.BlockSpec((1, 32), lambda i: (0, 0)),
                  pl.BlockSpec((1, 1), lambda i: (0, 0))],
        out_specs=pl.BlockSpec((1, EBLK), lambda i: (0, i)),
        out_shape=jax.ShapeDtypeStruct((1, EPAD), jnp.float32),
    )(at, p['W1'].T, p['b1'].reshape(32, 1), p['W2'].T, p['b2'].reshape(1, 1))
    return ea.reshape(EPAD)[:E]


def _gin_post_body(p_ref, g1_ref, be1_ref, w_ref, b_ref, g2_ref, be2_ref,
                   o_ref):
    agg = p_ref[:N]
    mu = jnp.mean(agg, axis=0, keepdims=True)
    var = jnp.mean((agg - mu) ** 2, axis=0, keepdims=True)
    h = jnp.maximum(
        g1_ref[...] * (agg - mu) / jnp.sqrt(var + 1e-5) + be1_ref[...], 0.0)
    h2 = _dot(h, w_ref[...]) + b_ref[...]
    mu2 = jnp.mean(h2, axis=0, keepdims=True)
    var2 = jnp.mean((h2 - mu2) ** 2, axis=0, keepdims=True)
    o_ref[...] = jnp.maximum(
        g2_ref[...] * (h2 - mu2) / jnp.sqrt(var2 + 1e-5) + be2_ref[...], 0.0)


def _gin_post(part, p):
    r1 = lambda a: a.reshape(1, HID)
    return pl.pallas_call(
        _gin_post_body,
        out_shape=jax.ShapeDtypeStruct((N, HID), jnp.float32),
        compiler_params=pltpu.CompilerParams(vmem_limit_bytes=100 * 1024 * 1024),
    )(part, r1(p['g1']), r1(p['be1']), p['W'], r1(p['b']),
      r1(p['g2']), r1(p['be2']))


def _pool_body(x_ref, gcol_ref, grow_ref, o_ref):
    y = x_ref[...]
    g = gcol_ref[...]
    k = 1
    while k < N:
        y_sh = jnp.concatenate(
            [jnp.full((k, HID), NEG_INF, jnp.float32), y[:-k]], axis=0)
        g_sh = jnp.concatenate(
            [jnp.full((k, 1), -1, jnp.int32), g[:-k]], axis=0)
        y = jnp.where(g == g_sh, jnp.maximum(y, y_sh), y)
        k *= 2
    grow = grow_ref[...]
    nxt = jnp.concatenate(
        [grow[:, 1:], jnp.full((1, 1), -1, jnp.int32)], axis=1)
    is_last = grow != nxt
    iota_b = lax.broadcasted_iota(jnp.int32, (B, 1), 0)
    eq = grow == iota_b
    sel = (eq & is_last).astype(jnp.float32)
    out = jnp.dot(sel, y, preferred_element_type=jnp.float32,
                  precision=lax.Precision.HIGHEST)
    cnt = jnp.sum(eq.astype(jnp.float32), axis=1, keepdims=True)
    o_ref[...] = jnp.where(cnt > 0, out, NEG_INF)


def _pool(x, gind):
    gind = gind.astype(jnp.int32)
    return pl.pallas_call(
        _pool_body,
        out_shape=jax.ShapeDtypeStruct((B, HID), jnp.float32),
        compiler_params=pltpu.CompilerParams(vmem_limit_bytes=100 * 1024 * 1024),
    )(x, gind.reshape(N, 1), gind.reshape(1, N))


def _bn(x, g, b):
    mu = jnp.mean(x, axis=0, keepdims=True)
    var = jnp.mean((x - mu) ** 2, axis=0, keepdims=True)
    return g * (x - mu) / jnp.sqrt(var + 1e-5) + b


def _gelu(x):
    return 0.5 * x * (1.0 + lax.erf(x / jnp.sqrt(jnp.float32(2.0))))


def _head_body(d_ref, pr_ref, fp_ref,
               fw1, fb1, fg1, fbe1, fw2, fb2, fg2, fbe2,
               wih, bih, whh, bhh,
               mw1, mb1, mw2, mb2, mw3, mb3, o_ref):
    dot = lambda a, b: _dot(a, b)
    f = jnp.maximum(_bn(dot(fp_ref[...], fw1[...]) + fb1[...],
                        fg1[...], fbe1[...]), 0.0)
    f = jnp.maximum(_bn(dot(f, fw2[...]) + fb2[...],
                        fg2[...], fbe2[...]), 0.0)
    gi = dot(d_ref[...], wih[...]) + bih[...]
    gh = dot(f, whh[...]) + bhh[...]
    r = jax.nn.sigmoid(gi[:, :HID] + gh[:, :HID])
    z = jax.nn.sigmoid(gi[:, HID:2 * HID] + gh[:, HID:2 * HID])
    nn_ = jnp.tanh(gi[:, 2 * HID:] + r * gh[:, 2 * HID:])
    h = (1.0 - z) * nn_ + z * f
    out = jnp.concatenate([h, pr_ref[...]], axis=1)
    h1 = _gelu(dot(out, mw1[...]) + mb1[...])
    h2 = _gelu(dot(h1, mw2[...]) + mb2[...])
    o_ref[...] = dot(h2, mw3[...]) + mb3[...]


def _head(d, pr, fp, params):
    fm = params['fp_mlp']
    g = params['gru']
    m = params['mlp']
    r1 = lambda a: a.reshape(1, -1)
    return pl.pallas_call(
        _head_body,
        out_shape=jax.ShapeDtypeStruct((B, 1), jnp.float32),
        compiler_params=pltpu.CompilerParams(vmem_limit_bytes=100 * 1024 * 1024),
    )(d, pr, fp,
      fm['W1'], r1(fm['b1']), r1(fm['g1']), r1(fm['be1']),
      fm['W2'], r1(fm['b2']), r1(fm['g2']), r1(fm['be2']),
      g['W_ih'], r1(g['b_ih']), g['W_hh'], r1(g['b_hh']),
      m['W1'], r1(m['b1']), m['W2'], r1(m['b2']), m['W3'], r1(m['b3']))


def _gnn(x, edge_index, edge_attr, gind, fcW, fcb, edge_p, layers):
    xin = _fc(x, fcW, fcb)
    ea = _edge_mlp(edge_attr, edge_p)
    src = edge_index[0].astype(jnp.int32)
    dst = edge_index[1].astype(jnp.int32)
    perm = jnp.argsort(dst // NODES_PER_W, stable=True)
    pad = EPAD_SC - E
    src_s = jnp.pad(src[perm], (0, pad)).reshape(CHUNKS, 1, CHUNK)
    dst_s = dst[perm]
    ea_s = jnp.pad(ea[perm], (0, pad)).reshape(CHUNKS, 1, CHUNK)
    bounds = jnp.arange(0, NW * NODES_PER_W, NODES_PER_W, dtype=jnp.int32)
    offs = jnp.searchsorted(dst_s // NODES_PER_W,
                            jnp.arange(NW, dtype=jnp.int32)).astype(jnp.int32)
    offs_end = jnp.concatenate([offs[1:], jnp.array([E], jnp.int32)])
    offs2 = jnp.pad(jnp.stack([offs, offs_end], axis=1),
                    ((0, 0), (0, L - 2))).reshape(NW, 1, L)
    dst_s = jnp.pad(dst_s, (0, pad)).reshape(CHUNKS, 1, CHUNK)
    zeros = jnp.zeros((NPAD, HID), jnp.float32)
    for p in layers:
        agg = _propagate(xin, src_s, dst_s, ea_s, offs2, zeros)
        xin = _gin_post(agg, p)
    return _pool(xin, gind)


def kernel(drug_x, drug_edge_index, drug_edge_attr, drug_graph_ind, fp_batch,
           prot_x, prot_edge_index, prot_edge_attr, prot_graph_ind, params):
    d = _gnn(drug_x, drug_edge_index, drug_edge_attr, drug_graph_ind,
             params['mol_fc_W'], params['mol_fc_b'], params['mol_edge'],
             params['mol_gnn'])
    pr = _gnn(prot_x, prot_edge_index, prot_edge_attr, prot_graph_ind,
              params['prot_fc_W'], params['prot_fc_b'], params['prot_edge'],
              params['prot_gnn'])
    out = _head(d, pr, fp_batch, params)
    return out.reshape(B)
